# Initial kernel scaffold; baseline (speedup 1.0000x reference)
import jax
import jax.numpy as jnp
from jax.experimental import pallas as pl


def kernel(x, WQ_w, WQ_b, WK_w, WK_b, fe_w, fe_b, V_w, V_b, U_w, U_b, aw_w, aw_b, clf_w, clf_b):
    raise NotImplementedError("write your pallas kernel here")


# speedup vs baseline: 5.5745x; 5.5745x over previous
import functools

import jax
import jax.numpy as jnp
from jax import lax
from jax.experimental import pallas as pl
from jax.experimental.pallas import tpu as pltpu

import numpy as np

HP = lax.Precision.HIGHEST
F32 = jnp.float32
I32MIN = np.int32(-2147483648)
I32MAX = np.int32(2147483647)
TOPK = 100


def _mm(a, b, dims):
    return lax.dot_general(a.astype(jnp.bfloat16), b.astype(jnp.bfloat16),
                           (dims, ((), ())), preferred_element_type=F32)


def _mm_f32(a, b, dims):
    return lax.dot_general(a, b, (dims, ((), ())), precision=HP,
                           preferred_element_type=F32)


def _proj_kernel(x_ref, w_ref, b_ref, o_ref):
    o_ref[...] = jnp.tanh(_mm(x_ref[...], w_ref[...], ((1,), (0,))) + b_ref[...])


def _skey(b):
    return b ^ ((b >> 31) & np.int32(0x7FFFFFFF))


def _sort_kernel(tq_ref, nq_ref, tk_ref, nk_ref, top_ref):
    qn = tq_ref[...] / nq_ref[...]
    kn = tk_ref[...] / nk_ref[...]
    A = _mm(qn, kn, ((1,), (1,)))
    R, N = A.shape
    lane = lax.broadcasted_iota(jnp.int32, (R, N), 1)
    x = A
    k = 2
    while k <= N:
        j = k // 2
        while j >= 1:
            jbit = (lane & j) != 0
            kbit = (lane & k) != 0
            up = pltpu.roll(x, N - j, 1)
            dn = pltpu.roll(x, j, 1)
            b = jnp.where(jbit, dn, up)
            x = jnp.where(jbit == kbit, jnp.maximum(x, b), jnp.minimum(x, b))
            j //= 2
        k *= 2
    top_ref[...] = x[:, :128]


def _rank_kernel(a1c_ref, a1r_ref, rank_ref):
    R = a1c_ref.shape[0]
    N = a1r_ref.shape[1]
    col = a1c_ref[...]
    row = a1r_ref[...]
    i0 = pl.program_id(0) * R
    ii = lax.broadcasted_iota(jnp.int32, (R, N), 0) + i0
    jj = lax.broadcasted_iota(jnp.int32, (R, N), 1)
    lt = (row < col).astype(jnp.int32)
    eq = ((row == col) & (jj < ii)).astype(jnp.int32)
    rank_ref[...] = jnp.sum(lt + eq, axis=1, keepdims=True)


def _attmil_kernel(n_sel, x_ref, few_ref, feb_ref, vw_ref, vb_ref, uw_ref,
                   ub_ref, aww_ref, awb_ref, clfw_ref, clfb_ref, rank_ref,
                   y_ref, aw_ref):
    xb = x_ref[0]
    h = jnp.maximum(_mm(xb, few_ref[...], ((1,), (0,))) + feb_ref[...], 0.0)
    av = jnp.tanh(_mm(h, vw_ref[...], ((1,), (0,))) + vb_ref[...])
    au = jax.nn.sigmoid(_mm(h, uw_ref[...], ((1,), (0,))) + ub_ref[...])
    s = _mm(av * au, aww_ref[...], ((1,), (0,))) + awb_ref[...]
    rank = rank_ref[...]
    selm = rank < n_sel
    sm = jnp.where(selm, s, -jnp.inf)
    m = jnp.max(sm, axis=0, keepdims=True)
    e = jnp.where(selm, jnp.exp(sm - m), 0.0)
    Z = jnp.sum(e, axis=0, keepdims=True)
    w = e / Z
    M = _mm(w, h, ((0,), (0,)))
    y = _mm(M, clfw_ref[...], ((1,), (0,))) + clfb_ref[...]
    y_ref[pl.ds(pl.program_id(0), 1), :] = y
    oh = (rank == lax.broadcasted_iota(jnp.int32, (rank.shape[0], n_sel), 1))
    aw = _mm_f32(w, oh.astype(F32), ((0,), (0,)))
    aw_ref[...] = aw[None]


def kernel(x, WQ_w, WQ_b, WK_w, WK_b, fe_w, fe_b, V_w, V_b, U_w, U_b,
           aw_w, aw_b, clf_w, clf_b):
    B, N, D = x.shape[1], x.shape[2], x.shape[3]
    HD = WQ_w.shape[1]
    HD2 = V_w.shape[1]
    n_sel = int(0.3 * N)
    RB = 256
    nblk = N // RB

    x0 = x[0]
    xq = x[0, 0]
    xk = x[1, 0]
    bq = WQ_b.reshape(1, HD)
    bk = WK_b.reshape(1, HD)

    def proj(xv, W, b):
        return pl.pallas_call(
            _proj_kernel,
            grid=(nblk,),
            in_specs=[
                pl.BlockSpec((RB, D), lambda i: (i, 0)),
                pl.BlockSpec((D, HD), lambda i: (0, 0)),
                pl.BlockSpec((1, HD), lambda i: (0, 0)),
            ],
            out_specs=pl.BlockSpec((RB, HD), lambda i: (i, 0)),
            out_shape=jax.ShapeDtypeStruct((N, HD), F32),
        )(xv, W, b)

    tq = proj(xq, WQ_w, bq)
    tk = proj(xk, WK_w, bk)
    nq = jnp.maximum(jnp.linalg.norm(tq, axis=1, keepdims=True), 1e-12)
    nk = jnp.maximum(jnp.linalg.norm(tk, axis=1, keepdims=True), 1e-12)

    top = pl.pallas_call(
        _sort_kernel,
        grid=(nblk,),
        in_specs=[
            pl.BlockSpec((RB, HD), lambda i: (i, 0)),
            pl.BlockSpec((RB, 1), lambda i: (i, 0)),
            pl.BlockSpec((N, HD), lambda i: (0, 0)),
            pl.BlockSpec((N, 1), lambda i: (0, 0)),
        ],
        out_specs=pl.BlockSpec((RB, 128), lambda i: (i, 0)),
        out_shape=jax.ShapeDtypeStruct((N, 128), F32),
    )(tq, nq, tk, nk)
    a1 = jnp.mean(top[:, :TOPK], axis=-1, keepdims=True)

    rank = pl.pallas_call(
        _rank_kernel,
        grid=(nblk,),
        in_specs=[
            pl.BlockSpec((RB, 1), lambda i: (i, 0)),
            pl.BlockSpec((1, N), lambda i: (0, 0)),
        ],
        out_specs=pl.BlockSpec((RB, 1), lambda i: (i, 0)),
        out_shape=jax.ShapeDtypeStruct((N, 1), jnp.int32),
    )(a1, a1.reshape(1, N))

    y, aw = pl.pallas_call(
        functools.partial(_attmil_kernel, n_sel),
        grid=(B,),
        in_specs=[
            pl.BlockSpec((1, N, D), lambda b: (b, 0, 0)),
            pl.BlockSpec((D, HD), lambda b: (0, 0)),
            pl.BlockSpec((1, HD), lambda b: (0, 0)),
            pl.BlockSpec((HD, HD2), lambda b: (0, 0)),
            pl.BlockSpec((1, HD2), lambda b: (0, 0)),
            pl.BlockSpec((HD, HD2), lambda b: (0, 0)),
            pl.BlockSpec((1, HD2), lambda b: (0, 0)),
            pl.BlockSpec((HD2, 1), lambda b: (0, 0)),
            pl.BlockSpec((1, 1), lambda b: (0, 0)),
            pl.BlockSpec((HD, 1), lambda b: (0, 0)),
            pl.BlockSpec((1, 1), lambda b: (0, 0)),
            pl.BlockSpec((N, 1), lambda b: (0, 0)),
        ],
        out_specs=[
            pl.BlockSpec((B, 1), lambda b: (0, 0)),
            pl.BlockSpec((1, 1, n_sel), lambda b: (b, 0, 0)),
        ],
        out_shape=[
            jax.ShapeDtypeStruct((B, 1), F32),
            jax.ShapeDtypeStruct((B, 1, n_sel), F32),
        ],
    )(x0, fe_w, fe_b.reshape(1, HD), V_w, V_b.reshape(1, HD2), U_w,
      U_b.reshape(1, HD2), aw_w, aw_b.reshape(1, 1), clf_w,
      clf_b.reshape(1, 1), rank)

    return (y, aw)
